# baseline (speedup 1.0000x reference)
import functools

import jax
import jax.numpy as jnp
import numpy as np
from jax.experimental import pallas as pl
from jax.experimental.pallas import tpu as pltpu

_D_MODEL = 256
_NUM_HEADS = 4
_HEAD_DIM = _D_MODEL // _NUM_HEADS
_TAU = 1.0
_GAMMA = 0.3
_BETA = 1.0
_ETA = 1.0
_NSEG = 8
_QB = 256
_KB = 256
_MINIT = -1e38
_QK_SCALE = _BETA / np.sqrt(_HEAD_DIM) / _TAU
_SIG_SCALE = 2.0 * _GAMMA / _TAU
_VW = 2 * _HEAD_DIM
_HD1 = _HEAD_DIM + 1


def _fused_body(qp_ref, kp_ref,
                phi_q_ref, sqb_ref,
                phi_k_ref, skb_ref,
                sk_ref, szk_ref,
                wa_ref, wb_ref, wv_ref,
                out_ref,
                pk_s, pv_s, ck_s):
    i = pl.program_id(0)
    qs = i * _QB
    f32 = jnp.float32
    bf16 = jnp.bfloat16
    dn_t = (((1,), (1,)), ((), ()))
    dn_m = (((1,), (0,)), ((), ()))
    nk = phi_k_ref.shape[0]

    @pl.when(i == 0)
    def _project_keys():
        def kinit(b, _):
            koff = b * _KB
            phik = phi_k_ref[pl.ds(koff, _KB), :]
            pk_s[pl.ds(koff, _KB), :] = jax.lax.dot_general(
                phik, wb_ref[...], dn_t,
                preferred_element_type=f32).astype(bf16)
            pv = jax.lax.dot_general(
                phik, wv_ref[...], dn_t,
                preferred_element_type=f32).astype(bf16)
            for h in range(_NUM_HEADS):
                pv_s[pl.ds(koff, _KB), h * _VW:h * _VW + _HEAD_DIM] = (
                    pv[:, h * _HEAD_DIM:(h + 1) * _HEAD_DIM])
                pv_s[pl.ds(koff, _KB),
                     h * _VW + _HEAD_DIM:h * _VW + _HD1] = (
                    jnp.ones((_KB, 1), bf16))
            sk = sk_ref[pl.ds(koff, _KB), :]
            ones_row = jnp.ones((1, sk.shape[1]), f32)
            kn = jax.lax.dot_general(ones_row, sk * sk, dn_t,
                                     preferred_element_type=f32)
            ck_s[:, pl.ds(koff, _KB)] = (
                -_GAMMA * kn + _ETA * jnp.log1p(szk_ref[:, pl.ds(koff, _KB)])
            ) / _TAU
            return 0
        jax.lax.fori_loop(0, nk // _KB, kinit, 0)

    pq = (jax.lax.dot_general(phi_q_ref[...], wa_ref[...], dn_t,
                              preferred_element_type=f32)
          * _QK_SCALE).astype(bf16)

    s0 = jnp.int32(0)
    s1 = jnp.int32(0)
    for j in range(1, _NSEG):
        s0 = s0 + (qp_ref[j] <= qs).astype(jnp.int32)
        s1 = s1 + (qp_ref[j] <= qs + _QB - 1).astype(jnp.int32)
    k_lo = kp_ref[s0]
    k_hi = kp_ref[s1 + 1]
    blo = k_lo // _KB
    bhi = (k_hi + _KB - 1) // _KB

    rows = qs + jax.lax.broadcasted_iota(jnp.int32, (_QB, 1), 0)
    seg_q = jnp.zeros((_QB, 1), jnp.int32)
    for j in range(1, _NSEG):
        seg_q = seg_q + (qp_ref[j] <= rows).astype(jnp.int32)

    sq = sqb_ref[...]

    def body(b, carry):
        ms, accs = carry
        koff = b * _KB
        pk = pk_s[pl.ds(koff, _KB), :]
        sk = skb_ref[pl.ds(koff, _KB), :]
        ck = ck_s[:, pl.ds(koff, _KB)]

        sigdot = jax.lax.dot_general(sq, sk, dn_t,
                                     preferred_element_type=f32)
        common = _SIG_SCALE * sigdot + ck

        cols = koff + jax.lax.broadcasted_iota(jnp.int32, (1, _KB), 1)
        seg_k = jnp.zeros((1, _KB), jnp.int32)
        for j in range(1, _NSEG):
            seg_k = seg_k + (kp_ref[j] <= cols).astype(jnp.int32)
        negmask = jnp.where(seg_q == seg_k, 0.0, -jnp.inf)
        common = common + negmask

        new_ms, new_accs = [], []
        for h in range(_NUM_HEADS):
            sl = slice(h * _HEAD_DIM, (h + 1) * _HEAD_DIM)
            s = common + jax.lax.dot_general(
                pq[:, sl], pk[:, sl], dn_t, preferred_element_type=f32)
            m_new = jnp.maximum(ms[h], jnp.max(s, axis=1, keepdims=True))
            p = jnp.exp(s - m_new)
            alpha = jnp.exp(ms[h] - m_new)
            res = jax.lax.dot_general(
                p.astype(bf16), pv_s[pl.ds(koff, _KB), h * _VW:h * _VW + _HD1],
                dn_m, preferred_element_type=f32)
            new_ms.append(m_new)
            new_accs.append(accs[h] * alpha + res)
        return tuple(new_ms), tuple(new_accs)

    m0 = tuple(jnp.full((_QB, 1), _MINIT, f32) for _ in range(_NUM_HEADS))
    a0 = tuple(jnp.zeros((_QB, _HD1), f32) for _ in range(_NUM_HEADS))
    ms, accs = jax.lax.fori_loop(blo, bhi, body, (m0, a0))

    for h in range(_NUM_HEADS):
        sl = slice(h * _HEAD_DIM, (h + 1) * _HEAD_DIM)
        out_ref[:, sl] = (accs[h][:, :_HEAD_DIM]
                          / jnp.maximum(accs[h][:, _HEAD_DIM:], 1e-20))


@functools.partial(jax.jit, static_argnames=("interpret",))
def _run(phi_q, sig_q, q_ptrs, phi_k, sig_k, size_k, k_ptrs, W_A, W_B, W_V,
         interpret=False):
    nq, d = phi_q.shape
    nk = phi_k.shape[0]
    dsig = sig_q.shape[1]
    nqb = nq // _QB
    szk2d = size_k.reshape(1, nk)
    bf16 = jnp.bfloat16

    grid_spec = pltpu.PrefetchScalarGridSpec(
        num_scalar_prefetch=2,
        grid=(nqb,),
        in_specs=[
            pl.BlockSpec((_QB, d), lambda i, qp, kp: (i, 0)),
            pl.BlockSpec((_QB, dsig), lambda i, qp, kp: (i, 0)),
            pl.BlockSpec((nk, d), lambda i, qp, kp: (0, 0)),
            pl.BlockSpec((nk, dsig), lambda i, qp, kp: (0, 0)),
            pl.BlockSpec((nk, dsig), lambda i, qp, kp: (0, 0)),
            pl.BlockSpec((1, nk), lambda i, qp, kp: (0, 0)),
            pl.BlockSpec((d, d), lambda i, qp, kp: (0, 0)),
            pl.BlockSpec((d, d), lambda i, qp, kp: (0, 0)),
            pl.BlockSpec((d, d), lambda i, qp, kp: (0, 0)),
        ],
        out_specs=pl.BlockSpec((_QB, d), lambda i, qp, kp: (i, 0)),
        scratch_shapes=[
            pltpu.VMEM((nk, d), bf16),
            pltpu.VMEM((nk, _NUM_HEADS * _VW), bf16),
            pltpu.VMEM((1, nk), jnp.float32),
        ],
    )
    out = pl.pallas_call(
        _fused_body,
        grid_spec=grid_spec,
        out_shape=jax.ShapeDtypeStruct((nq, d), jnp.float32),
        compiler_params=pltpu.CompilerParams(
            dimension_semantics=("arbitrary",)),
        interpret=interpret,
    )(q_ptrs, k_ptrs, phi_q.astype(bf16), sig_q.astype(bf16),
      phi_k.astype(bf16), sig_k.astype(bf16), sig_k, szk2d,
      W_A.astype(bf16), W_B.astype(bf16), W_V.astype(bf16))
    return out


def kernel(phi_q, sig_q, size_q, q_ptrs, phi_k, sig_k, size_k, k_ptrs,
           W_A, W_B, W_V):
    out = _run(phi_q, sig_q, q_ptrs, phi_k, sig_k, size_k, k_ptrs,
               W_A, W_B, W_V)
    nq = phi_q.shape[0]
    return (out.reshape(nq, _NUM_HEADS, _HEAD_DIM), q_ptrs)


# speedup vs baseline: 1.1126x; 1.1126x over previous
import functools

import jax
import jax.numpy as jnp
import numpy as np
from jax.experimental import pallas as pl
from jax.experimental.pallas import tpu as pltpu

_D_MODEL = 256
_NUM_HEADS = 4
_HEAD_DIM = _D_MODEL // _NUM_HEADS
_TAU = 1.0
_GAMMA = 0.3
_BETA = 1.0
_ETA = 1.0
_NSEG = 8
_QB = 256
_KB = 256
_MINIT = -1e38
_QK_SCALE = _BETA / np.sqrt(_HEAD_DIM) / _TAU
_SIG_SCALE = 2.0 * _GAMMA / _TAU
_VW = 2 * _HEAD_DIM
_HD1 = _HEAD_DIM + 1


def _fused_body(qp_ref, kp_ref,
                phi_q_ref, sqb_ref,
                phi_k_ref, skb_ref,
                sk_ref, szk_ref,
                wa_ref, wb_ref, wv_ref,
                out_ref,
                pk_s, pv_s, ck_s):
    i = pl.program_id(0)
    qs = i * _QB
    f32 = jnp.float32
    bf16 = jnp.bfloat16
    dn_t = (((1,), (1,)), ((), ()))
    dn_m = (((1,), (0,)), ((), ()))
    nk = phi_k_ref.shape[0]

    @pl.when(i == 0)
    def _project_keys():
        def kinit(b, _):
            koff = b * _KB
            phik = phi_k_ref[pl.ds(koff, _KB), :]
            pk_s[pl.ds(koff, _KB), :] = jax.lax.dot_general(
                phik, wb_ref[...], dn_t,
                preferred_element_type=f32).astype(bf16)
            pv = jax.lax.dot_general(
                phik, wv_ref[...], dn_t,
                preferred_element_type=f32).astype(bf16)
            for h in range(_NUM_HEADS):
                pv_s[pl.ds(koff, _KB), h * _VW:h * _VW + _HEAD_DIM] = (
                    pv[:, h * _HEAD_DIM:(h + 1) * _HEAD_DIM])
                pv_s[pl.ds(koff, _KB),
                     h * _VW + _HEAD_DIM:h * _VW + _HD1] = (
                    jnp.ones((_KB, 1), bf16))
            sk = sk_ref[pl.ds(koff, _KB), :]
            ones_row = jnp.ones((1, sk.shape[1]), f32)
            kn = jax.lax.dot_general(ones_row, sk * sk, dn_t,
                                     preferred_element_type=f32)
            ck_s[:, pl.ds(koff, _KB)] = (
                -_GAMMA * kn + _ETA * jnp.log1p(szk_ref[:, pl.ds(koff, _KB)])
            ) / _TAU
            return 0
        jax.lax.fori_loop(0, nk // _KB, kinit, 0)

    pq = (jax.lax.dot_general(phi_q_ref[...], wa_ref[...], dn_t,
                              preferred_element_type=f32)
          * _QK_SCALE).astype(bf16)

    s0 = jnp.int32(0)
    s1 = jnp.int32(0)
    for j in range(1, _NSEG):
        s0 = s0 + (qp_ref[j] <= qs).astype(jnp.int32)
        s1 = s1 + (qp_ref[j] <= qs + _QB - 1).astype(jnp.int32)
    k_lo = kp_ref[s0]
    k_hi = kp_ref[s1 + 1]
    blo = k_lo // _KB
    bhi = (k_hi + _KB - 1) // _KB

    rows = qs + jax.lax.broadcasted_iota(jnp.int32, (1, _QB), 1)
    segq_row = jnp.zeros((1, _QB), jnp.int32)
    for j in range(1, _NSEG):
        segq_row = segq_row + (qp_ref[j] <= rows).astype(jnp.int32)
    seg_q = segq_row.reshape(_QB, 1)

    sq = sqb_ref[...]

    def body(b, carry):
        ms, accs = carry
        koff = b * _KB
        pk = pk_s[pl.ds(koff, _KB), :]
        sk = skb_ref[pl.ds(koff, _KB), :]
        ck = ck_s[:, pl.ds(koff, _KB)]

        sigdot = jax.lax.dot_general(sq, sk, dn_t,
                                     preferred_element_type=f32)

        cols = koff + jax.lax.broadcasted_iota(jnp.int32, (1, _KB), 1)
        seg_k = jnp.zeros((1, _KB), jnp.int32)
        for j in range(1, _NSEG):
            seg_k = seg_k + (kp_ref[j] <= cols).astype(jnp.int32)
        common = jnp.where(seg_q == seg_k, ck, -jnp.inf) + sigdot

        dots = [jax.lax.dot_general(
            pq[:, h * _HEAD_DIM:(h + 1) * _HEAD_DIM],
            pk[:, h * _HEAD_DIM:(h + 1) * _HEAD_DIM],
            dn_t, preferred_element_type=f32) for h in range(_NUM_HEADS)]

        new_ms, new_accs = [], []
        for h in range(_NUM_HEADS):
            s = common + dots[h]
            m_new = jnp.maximum(ms[h], jnp.max(s, axis=1, keepdims=True))
            p = jnp.exp(s - m_new)
            alpha = jnp.exp(ms[h] - m_new)
            res = jax.lax.dot_general(
                p.astype(bf16), pv_s[pl.ds(koff, _KB), h * _VW:h * _VW + _HD1],
                dn_m, preferred_element_type=f32)
            new_ms.append(m_new)
            new_accs.append(accs[h] * alpha + res)
        return tuple(new_ms), tuple(new_accs)

    m0 = tuple(jnp.full((_QB, 1), _MINIT, f32) for _ in range(_NUM_HEADS))
    a0 = tuple(jnp.zeros((_QB, _HD1), f32) for _ in range(_NUM_HEADS))
    ms, accs = jax.lax.fori_loop(blo, bhi, body, (m0, a0))

    for h in range(_NUM_HEADS):
        sl = slice(h * _HEAD_DIM, (h + 1) * _HEAD_DIM)
        inv = 1.0 / jnp.maximum(accs[h][:, _HEAD_DIM:], 1e-20)
        out_ref[:, sl] = accs[h][:, :_HEAD_DIM] * inv


@functools.partial(jax.jit, static_argnames=("interpret",))
def _run(phi_q, sig_q, q_ptrs, phi_k, sig_k, size_k, k_ptrs, W_A, W_B, W_V,
         interpret=False):
    nq, d = phi_q.shape
    nk = phi_k.shape[0]
    dsig = sig_q.shape[1]
    nqb = nq // _QB
    szk2d = size_k.reshape(1, nk)
    bf16 = jnp.bfloat16

    grid_spec = pltpu.PrefetchScalarGridSpec(
        num_scalar_prefetch=2,
        grid=(nqb,),
        in_specs=[
            pl.BlockSpec((_QB, d), lambda i, qp, kp: (i, 0)),
            pl.BlockSpec((_QB, dsig), lambda i, qp, kp: (i, 0)),
            pl.BlockSpec((nk, d), lambda i, qp, kp: (0, 0)),
            pl.BlockSpec((nk, dsig), lambda i, qp, kp: (0, 0)),
            pl.BlockSpec((nk, dsig), lambda i, qp, kp: (0, 0)),
            pl.BlockSpec((1, nk), lambda i, qp, kp: (0, 0)),
            pl.BlockSpec((d, d), lambda i, qp, kp: (0, 0)),
            pl.BlockSpec((d, d), lambda i, qp, kp: (0, 0)),
            pl.BlockSpec((d, d), lambda i, qp, kp: (0, 0)),
        ],
        out_specs=pl.BlockSpec((_QB, d), lambda i, qp, kp: (i, 0)),
        scratch_shapes=[
            pltpu.VMEM((nk, d), bf16),
            pltpu.VMEM((nk, _NUM_HEADS * _VW), bf16),
            pltpu.VMEM((1, nk), jnp.float32),
        ],
    )
    out = pl.pallas_call(
        _fused_body,
        grid_spec=grid_spec,
        out_shape=jax.ShapeDtypeStruct((nq, d), jnp.float32),
        compiler_params=pltpu.CompilerParams(
            dimension_semantics=("arbitrary",)),
        interpret=interpret,
    )(q_ptrs, k_ptrs, phi_q.astype(bf16), (sig_q * _SIG_SCALE).astype(bf16),
      phi_k.astype(bf16), sig_k.astype(bf16), sig_k, szk2d,
      W_A.astype(bf16), W_B.astype(bf16), W_V.astype(bf16))
    return out


def kernel(phi_q, sig_q, size_q, q_ptrs, phi_k, sig_k, size_k, k_ptrs,
           W_A, W_B, W_V):
    out = _run(phi_q, sig_q, q_ptrs, phi_k, sig_k, size_k, k_ptrs,
               W_A, W_B, W_V)
    nq = phi_q.shape[0]
    return (out.reshape(nq, _NUM_HEADS, _HEAD_DIM), q_ptrs)


# speedup vs baseline: 1.1129x; 1.0003x over previous
import functools

import jax
import jax.numpy as jnp
import numpy as np
from jax.experimental import pallas as pl
from jax.experimental.pallas import tpu as pltpu

_D_MODEL = 256
_NUM_HEADS = 4
_HEAD_DIM = _D_MODEL // _NUM_HEADS
_TAU = 1.0
_GAMMA = 0.3
_BETA = 1.0
_ETA = 1.0
_NSEG = 8
_QB = 256
_KB = 256
_MINIT = -1e38
_QK_SCALE = _BETA / np.sqrt(_HEAD_DIM) / _TAU
_SIG_SCALE = 2.0 * _GAMMA / _TAU
_VW = 2 * _HEAD_DIM
_HD1 = _HEAD_DIM + 1


def _fused_body(qp_ref, kp_ref,
                phi_q_ref, sqb_ref,
                phi_k_ref, skb_ref,
                sk_ref, szk_ref,
                wa_ref, wb_ref, wv_ref,
                out_ref,
                pk_s, pv_s, ck_s):
    i = pl.program_id(0)
    qs = i * _QB
    f32 = jnp.float32
    bf16 = jnp.bfloat16
    dn_t = (((1,), (1,)), ((), ()))
    dn_m = (((1,), (0,)), ((), ()))
    nk = phi_k_ref.shape[0]

    @pl.when(i == 0)
    def _project_keys():
        def kinit(b, _):
            koff = b * _KB
            phik = phi_k_ref[pl.ds(koff, _KB), :]
            pk_s[pl.ds(koff, _KB), :] = jax.lax.dot_general(
                phik, wb_ref[...], dn_t,
                preferred_element_type=f32).astype(bf16)
            pv = jax.lax.dot_general(
                phik, wv_ref[...], dn_t,
                preferred_element_type=f32).astype(bf16)
            ones_blk = jnp.ones((_KB, _HEAD_DIM), bf16)
            for h in range(_NUM_HEADS):
                pv_s[pl.ds(koff, _KB), h * _VW:h * _VW + _HEAD_DIM] = (
                    pv[:, h * _HEAD_DIM:(h + 1) * _HEAD_DIM])
                pv_s[pl.ds(koff, _KB),
                     h * _VW + _HEAD_DIM:(h + 1) * _VW] = ones_blk
            sk = sk_ref[pl.ds(koff, _KB), :]
            ones_row = jnp.ones((1, sk.shape[1]), f32)
            kn = jax.lax.dot_general(ones_row, sk * sk, dn_t,
                                     preferred_element_type=f32)
            ck_s[:, pl.ds(koff, _KB)] = (
                -_GAMMA * kn + _ETA * jnp.log1p(szk_ref[:, pl.ds(koff, _KB)])
            ) / _TAU
            return 0
        jax.lax.fori_loop(0, nk // _KB, kinit, 0)

    rows = qs + jax.lax.broadcasted_iota(jnp.int32, (1, _QB), 1)
    segq_row = jnp.zeros((1, _QB), jnp.int32)
    for j in range(1, _NSEG):
        segq_row = segq_row + (qp_ref[j] <= rows).astype(jnp.int32)
    seg_q = segq_row.reshape(_QB, 1)

    s0 = jnp.int32(0)
    s1 = jnp.int32(0)
    for j in range(1, _NSEG):
        s0 = s0 + (qp_ref[j] <= qs).astype(jnp.int32)
        s1 = s1 + (qp_ref[j] <= qs + _QB - 1).astype(jnp.int32)
    k_lo = kp_ref[s0]
    k_hi = kp_ref[s1 + 1]
    blo = k_lo // _KB
    bhi = (k_hi + _KB - 1) // _KB

    pq = (jax.lax.dot_general(phi_q_ref[...], wa_ref[...], dn_t,
                              preferred_element_type=f32)
          * _QK_SCALE).astype(bf16)

    sq = sqb_ref[...]

    def body(b, carry):
        ms, accs = carry
        koff = b * _KB
        pk = pk_s[pl.ds(koff, _KB), :]
        sk = skb_ref[pl.ds(koff, _KB), :]
        ck = ck_s[:, pl.ds(koff, _KB)]

        sigdot = jax.lax.dot_general(sq, sk, dn_t,
                                     preferred_element_type=f32)

        cols = koff + jax.lax.broadcasted_iota(jnp.int32, (1, _KB), 1)
        seg_k = jnp.zeros((1, _KB), jnp.int32)
        for j in range(1, _NSEG):
            seg_k = seg_k + (kp_ref[j] <= cols).astype(jnp.int32)
        common = jnp.where(seg_q == seg_k, ck, -jnp.inf) + sigdot

        dots = [jax.lax.dot_general(
            pq[:, h * _HEAD_DIM:(h + 1) * _HEAD_DIM],
            pk[:, h * _HEAD_DIM:(h + 1) * _HEAD_DIM],
            dn_t, preferred_element_type=f32) for h in range(_NUM_HEADS)]

        new_ms, new_accs = [], []
        for h in range(_NUM_HEADS):
            s = common + dots[h]
            m_new = jnp.maximum(ms[h], jnp.max(s, axis=1, keepdims=True))
            p = jnp.exp(s - m_new)
            alpha = jnp.exp(ms[h] - m_new)
            res = jax.lax.dot_general(
                p.astype(bf16), pv_s[pl.ds(koff, _KB), h * _VW:h * _VW + _HD1],
                dn_m, preferred_element_type=f32)
            new_ms.append(m_new)
            new_accs.append(accs[h] * alpha + res)
        return tuple(new_ms), tuple(new_accs)

    m0 = tuple(jnp.full((_QB, 1), _MINIT, f32) for _ in range(_NUM_HEADS))
    a0 = tuple(jnp.zeros((_QB, _HD1), f32) for _ in range(_NUM_HEADS))
    ms, accs = jax.lax.fori_loop(blo, bhi, body, (m0, a0))

    for h in range(_NUM_HEADS):
        sl = slice(h * _HEAD_DIM, (h + 1) * _HEAD_DIM)
        inv = 1.0 / jnp.maximum(accs[h][:, _HEAD_DIM:], 1e-20)
        out_ref[:, sl] = accs[h][:, :_HEAD_DIM] * inv


@functools.partial(jax.jit, static_argnames=("interpret",))
def _run(phi_q, sig_q, q_ptrs, phi_k, sig_k, size_k, k_ptrs, W_A, W_B, W_V,
         interpret=False):
    nq, d = phi_q.shape
    nk = phi_k.shape[0]
    dsig = sig_q.shape[1]
    nqb = nq // _QB
    szk2d = size_k.reshape(1, nk)
    bf16 = jnp.bfloat16

    grid_spec = pltpu.PrefetchScalarGridSpec(
        num_scalar_prefetch=2,
        grid=(nqb,),
        in_specs=[
            pl.BlockSpec((_QB, d), lambda i, qp, kp: (i, 0)),
            pl.BlockSpec((_QB, dsig), lambda i, qp, kp: (i, 0)),
            pl.BlockSpec((nk, d), lambda i, qp, kp: (0, 0)),
            pl.BlockSpec((nk, dsig), lambda i, qp, kp: (0, 0)),
            pl.BlockSpec((nk, dsig), lambda i, qp, kp: (0, 0)),
            pl.BlockSpec((1, nk), lambda i, qp, kp: (0, 0)),
            pl.BlockSpec((d, d), lambda i, qp, kp: (0, 0)),
            pl.BlockSpec((d, d), lambda i, qp, kp: (0, 0)),
            pl.BlockSpec((d, d), lambda i, qp, kp: (0, 0)),
        ],
        out_specs=pl.BlockSpec((_QB, d), lambda i, qp, kp: (i, 0)),
        scratch_shapes=[
            pltpu.VMEM((nk, d), bf16),
            pltpu.VMEM((nk, _NUM_HEADS * _VW), bf16),
            pltpu.VMEM((1, nk), jnp.float32),
        ],
    )
    out = pl.pallas_call(
        _fused_body,
        grid_spec=grid_spec,
        out_shape=jax.ShapeDtypeStruct((nq, d), jnp.float32),
        compiler_params=pltpu.CompilerParams(
            dimension_semantics=("arbitrary",)),
        interpret=interpret,
    )(q_ptrs, k_ptrs, phi_q.astype(bf16), (sig_q * _SIG_SCALE).astype(bf16),
      phi_k.astype(bf16), sig_k.astype(bf16), sig_k, szk2d,
      W_A.astype(bf16), W_B.astype(bf16), W_V.astype(bf16))
    return out


def kernel(phi_q, sig_q, size_q, q_ptrs, phi_k, sig_k, size_k, k_ptrs,
           W_A, W_B, W_V):
    out = _run(phi_q, sig_q, q_ptrs, phi_k, sig_k, size_k, k_ptrs,
               W_A, W_B, W_V)
    nq = phi_q.shape[0]
    return (out.reshape(nq, _NUM_HEADS, _HEAD_DIM), q_ptrs)


# speedup vs baseline: 1.1621x; 1.0443x over previous
import functools

import jax
import jax.numpy as jnp
import numpy as np
from jax.experimental import pallas as pl
from jax.experimental.pallas import tpu as pltpu

_D_MODEL = 256
_NUM_HEADS = 4
_HEAD_DIM = _D_MODEL // _NUM_HEADS
_TAU = 1.0
_GAMMA = 0.3
_BETA = 1.0
_ETA = 1.0
_NSEG = 8
_QB = 256
_KB = 256
_MINIT = -1e38
_QK_SCALE = _BETA / np.sqrt(_HEAD_DIM) / _TAU
_SIG_SCALE = 2.0 * _GAMMA / _TAU
_VW = 2 * _HEAD_DIM
_HD1 = _HEAD_DIM + 1


def _fused_body(qp_ref, kp_ref,
                phi_q_ref, sqb_ref,
                phi_k_ref, skb_ref,
                sk_ref, szk_ref,
                wa_ref, wb_ref, wv_ref,
                out_ref,
                pk_s, pv_s, ck_s, bnd_s):
    i = pl.program_id(0)
    qs = i * _QB
    f32 = jnp.float32
    bf16 = jnp.bfloat16
    dn_t = (((1,), (1,)), ((), ()))
    dn_m = (((1,), (0,)), ((), ()))
    nk = phi_k_ref.shape[0]

    hrow = jax.lax.broadcasted_iota(jnp.int32, (_D_MODEL, _NUM_HEADS), 0)
    hcol = jax.lax.broadcasted_iota(jnp.int32, (_D_MODEL, _NUM_HEADS), 1)
    hsel = (hrow // _HEAD_DIM == hcol).astype(bf16)
    ones16 = jnp.ones((1, 16), bf16)

    @pl.when(i == 0)
    def _project_keys():
        def kinit(b, carry):
            kmax2, smax2, ckmax = carry
            koff = b * _KB
            phik = phi_k_ref[pl.ds(koff, _KB), :]
            pkf = jax.lax.dot_general(phik, wb_ref[...], dn_t,
                                      preferred_element_type=f32)
            pk_s[pl.ds(koff, _KB), :] = pkf.astype(bf16)
            kn2 = jax.lax.dot_general((pkf * pkf).astype(bf16), hsel, dn_m,
                                      preferred_element_type=f32)
            kmax2 = jnp.maximum(kmax2, jnp.max(kn2, axis=0, keepdims=True))
            pv = jax.lax.dot_general(
                phik, wv_ref[...], dn_t,
                preferred_element_type=f32).astype(bf16)
            ones_blk = jnp.ones((_KB, _HEAD_DIM), bf16)
            for h in range(_NUM_HEADS):
                pv_s[pl.ds(koff, _KB), h * _VW:h * _VW + _HEAD_DIM] = (
                    pv[:, h * _HEAD_DIM:(h + 1) * _HEAD_DIM])
                pv_s[pl.ds(koff, _KB),
                     h * _VW + _HEAD_DIM:(h + 1) * _VW] = ones_blk
            sk = sk_ref[pl.ds(koff, _KB), :]
            sn2 = jnp.sum(sk * sk, axis=1, keepdims=True)
            smax2 = jnp.maximum(smax2, jnp.max(sn2, axis=0, keepdims=True))
            ones_row = jnp.ones((1, sk.shape[1]), f32)
            kn = jax.lax.dot_general(ones_row, sk * sk, dn_t,
                                     preferred_element_type=f32)
            ckrow = (-_GAMMA * kn
                     + _ETA * jnp.log1p(szk_ref[:, pl.ds(koff, _KB)])) / _TAU
            ck_s[:, pl.ds(koff, _KB)] = ckrow
            ckmax = jnp.maximum(ckmax, jnp.max(ckrow, axis=1, keepdims=True))
            return kmax2, smax2, ckmax
        kmax2, smax2, ckmax = jax.lax.fori_loop(
            0, nk // _KB, kinit,
            (jnp.zeros((1, _NUM_HEADS), f32), jnp.zeros((1, 1), f32),
             jnp.full((1, 1), _MINIT, f32)))
        bnd_s[:, 0:_NUM_HEADS] = kmax2
        bnd_s[:, _NUM_HEADS:_NUM_HEADS + 1] = smax2
        bnd_s[:, _NUM_HEADS + 1:_NUM_HEADS + 2] = ckmax

    rows = qs + jax.lax.broadcasted_iota(jnp.int32, (1, _QB), 1)
    segq_row = jnp.zeros((1, _QB), jnp.int32)
    for j in range(1, _NSEG):
        segq_row = segq_row + (qp_ref[j] <= rows).astype(jnp.int32)
    seg_q = segq_row.reshape(_QB, 1)

    s0 = jnp.int32(0)
    s1 = jnp.int32(0)
    for j in range(1, _NSEG):
        s0 = s0 + (qp_ref[j] <= qs).astype(jnp.int32)
        s1 = s1 + (qp_ref[j] <= qs + _QB - 1).astype(jnp.int32)
    k_lo = kp_ref[s0]
    k_hi = kp_ref[s1 + 1]
    blo = k_lo // _KB
    bhi = (k_hi + _KB - 1) // _KB

    pqf = (jax.lax.dot_general(phi_q_ref[...], wa_ref[...], dn_t,
                               preferred_element_type=f32)
           * _QK_SCALE)
    pq = pqf.astype(bf16)

    sq = sqb_ref[...]

    kmax2 = bnd_s[:, 0:_NUM_HEADS]
    smax2 = bnd_s[:, _NUM_HEADS:_NUM_HEADS + 1]
    ckmax = bnd_s[:, _NUM_HEADS + 1:_NUM_HEADS + 2]
    qn2 = jax.lax.dot_general((pqf * pqf).astype(bf16), hsel, dn_m,
                              preferred_element_type=f32)
    sqf = sq.astype(f32)
    sqn2 = jnp.sum(sqf * sqf, axis=1, keepdims=True)
    bb = (jnp.sqrt(qn2 * kmax2) * 1.01
          + jnp.sqrt(sqn2 * smax2) * 1.01 + ckmax)
    bhs = [bb[:, h:h + 1] for h in range(_NUM_HEADS)]

    def body(b, accs):
        koff = b * _KB
        pk = pk_s[pl.ds(koff, _KB), :]
        sk = skb_ref[pl.ds(koff, _KB), :]
        ck = ck_s[:, pl.ds(koff, _KB)]

        sigdot = jax.lax.dot_general(sq, sk, dn_t,
                                     preferred_element_type=f32)

        cols = koff + jax.lax.broadcasted_iota(jnp.int32, (1, _KB), 1)
        seg_k = jnp.zeros((1, _KB), jnp.int32)
        for j in range(1, _NSEG):
            seg_k = seg_k + (kp_ref[j] <= cols).astype(jnp.int32)
        common = jnp.where(seg_q == seg_k, ck, -jnp.inf) + sigdot

        dots = [jax.lax.dot_general(
            pq[:, h * _HEAD_DIM:(h + 1) * _HEAD_DIM],
            pk[:, h * _HEAD_DIM:(h + 1) * _HEAD_DIM],
            dn_t, preferred_element_type=f32) for h in range(_NUM_HEADS)]

        new_accs = []
        for h in range(_NUM_HEADS):
            p = jnp.exp(common + dots[h] - bhs[h])
            res = jax.lax.dot_general(
                p.astype(bf16), pv_s[pl.ds(koff, _KB), h * _VW:h * _VW + _HD1],
                dn_m, preferred_element_type=f32)
            new_accs.append(accs[h] + res)
        return tuple(new_accs)

    a0 = tuple(jnp.zeros((_QB, _HD1), f32) for _ in range(_NUM_HEADS))
    accs = jax.lax.fori_loop(blo, bhi, body, a0)

    for h in range(_NUM_HEADS):
        sl = slice(h * _HEAD_DIM, (h + 1) * _HEAD_DIM)
        inv = 1.0 / jnp.maximum(accs[h][:, _HEAD_DIM:], 1e-20)
        out_ref[:, sl] = accs[h][:, :_HEAD_DIM] * inv


@functools.partial(jax.jit, static_argnames=("interpret",))
def _run(phi_q, sig_q, q_ptrs, phi_k, sig_k, size_k, k_ptrs, W_A, W_B, W_V,
         interpret=False):
    nq, d = phi_q.shape
    nk = phi_k.shape[0]
    dsig = sig_q.shape[1]
    nqb = nq // _QB
    szk2d = size_k.reshape(1, nk)
    bf16 = jnp.bfloat16

    grid_spec = pltpu.PrefetchScalarGridSpec(
        num_scalar_prefetch=2,
        grid=(nqb,),
        in_specs=[
            pl.BlockSpec((_QB, d), lambda i, qp, kp: (i, 0)),
            pl.BlockSpec((_QB, dsig), lambda i, qp, kp: (i, 0)),
            pl.BlockSpec((nk, d), lambda i, qp, kp: (0, 0)),
            pl.BlockSpec((nk, dsig), lambda i, qp, kp: (0, 0)),
            pl.BlockSpec((nk, dsig), lambda i, qp, kp: (0, 0)),
            pl.BlockSpec((1, nk), lambda i, qp, kp: (0, 0)),
            pl.BlockSpec((d, d), lambda i, qp, kp: (0, 0)),
            pl.BlockSpec((d, d), lambda i, qp, kp: (0, 0)),
            pl.BlockSpec((d, d), lambda i, qp, kp: (0, 0)),
        ],
        out_specs=pl.BlockSpec((_QB, d), lambda i, qp, kp: (i, 0)),
        scratch_shapes=[
            pltpu.VMEM((nk, d), bf16),
            pltpu.VMEM((nk, _NUM_HEADS * _VW), bf16),
            pltpu.VMEM((1, nk), jnp.float32),
            pltpu.VMEM((1, 128), jnp.float32),
        ],
    )
    out = pl.pallas_call(
        _fused_body,
        grid_spec=grid_spec,
        out_shape=jax.ShapeDtypeStruct((nq, d), jnp.float32),
        compiler_params=pltpu.CompilerParams(
            dimension_semantics=("arbitrary",)),
        interpret=interpret,
    )(q_ptrs, k_ptrs, phi_q.astype(bf16), (sig_q * _SIG_SCALE).astype(bf16),
      phi_k.astype(bf16), sig_k.astype(bf16), sig_k, szk2d,
      W_A.astype(bf16), W_B.astype(bf16), W_V.astype(bf16))
    return out


def kernel(phi_q, sig_q, size_q, q_ptrs, phi_k, sig_k, size_k, k_ptrs,
           W_A, W_B, W_V):
    out = _run(phi_q, sig_q, q_ptrs, phi_k, sig_k, size_k, k_ptrs,
               W_A, W_B, W_V)
    nq = phi_q.shape[0]
    return (out.reshape(nq, _NUM_HEADS, _HEAD_DIM), q_ptrs)


# speedup vs baseline: 1.2963x; 1.1154x over previous
import functools

import jax
import jax.numpy as jnp
import numpy as np
from jax.experimental import pallas as pl
from jax.experimental.pallas import tpu as pltpu

_D_MODEL = 256
_NUM_HEADS = 4
_HEAD_DIM = _D_MODEL // _NUM_HEADS
_TAU = 1.0
_GAMMA = 0.3
_BETA = 1.0
_ETA = 1.0
_NSEG = 8
_QB = 256
_KB = 256
_MINIT = -1e38
_QK_SCALE = _BETA / np.sqrt(_HEAD_DIM) / _TAU
_SIG_SCALE = 2.0 * _GAMMA / _TAU
_VW = 2 * _HEAD_DIM
_HD1 = _HEAD_DIM + 1


def _fused_body(qp_ref, kp_ref,
                phi_q_ref, sq_ref,
                phi_k_ref, sk_ref,
                szk_ref,
                wa_ref, wb_ref, wv_ref,
                out_ref,
                pk_s, pv_s, ck_s, bnd_s):
    i = pl.program_id(0)
    qs = i * _QB
    f32 = jnp.float32
    bf16 = jnp.bfloat16
    dn_t = (((1,), (1,)), ((), ()))
    dn_m = (((1,), (0,)), ((), ()))
    nk = phi_k_ref.shape[0]

    hrow = jax.lax.broadcasted_iota(jnp.int32, (_D_MODEL, _NUM_HEADS), 0)
    hcol = jax.lax.broadcasted_iota(jnp.int32, (_D_MODEL, _NUM_HEADS), 1)
    hsel = (hrow // _HEAD_DIM == hcol).astype(bf16)

    @pl.when(i == 0)
    def _project_keys():
        wbb = wb_ref[...].astype(bf16)
        wvb = wv_ref[...].astype(bf16)

        def kinit(b, carry):
            kmax2, smax2, ckmax = carry
            koff = b * _KB
            phik = phi_k_ref[pl.ds(koff, _KB), :].astype(bf16)
            pkf = jax.lax.dot_general(phik, wbb, dn_t,
                                      preferred_element_type=f32)
            pk_s[pl.ds(koff, _KB), :] = pkf.astype(bf16)
            kn2 = jax.lax.dot_general((pkf * pkf).astype(bf16), hsel, dn_m,
                                      preferred_element_type=f32)
            kmax2 = jnp.maximum(kmax2, jnp.max(kn2, axis=0, keepdims=True))
            pv = jax.lax.dot_general(
                phik, wvb, dn_t,
                preferred_element_type=f32).astype(bf16)
            ones_blk = jnp.ones((_KB, _HEAD_DIM), bf16)
            for h in range(_NUM_HEADS):
                pv_s[pl.ds(koff, _KB), h * _VW:h * _VW + _HEAD_DIM] = (
                    pv[:, h * _HEAD_DIM:(h + 1) * _HEAD_DIM])
                pv_s[pl.ds(koff, _KB),
                     h * _VW + _HEAD_DIM:(h + 1) * _VW] = ones_blk
            sk = sk_ref[pl.ds(koff, _KB), :]
            sn2 = jnp.sum(sk * sk, axis=1, keepdims=True)
            smax2 = jnp.maximum(smax2, jnp.max(sn2, axis=0, keepdims=True))
            ones_row = jnp.ones((1, sk.shape[1]), f32)
            kn = jax.lax.dot_general(ones_row, sk * sk, dn_t,
                                     preferred_element_type=f32)
            ckrow = (-_GAMMA * kn
                     + _ETA * jnp.log1p(szk_ref[:, pl.ds(koff, _KB)])) / _TAU
            ck_s[:, pl.ds(koff, _KB)] = ckrow
            ckmax = jnp.maximum(ckmax, jnp.max(ckrow, axis=1, keepdims=True))
            return kmax2, smax2, ckmax
        kmax2, smax2, ckmax = jax.lax.fori_loop(
            0, nk // _KB, kinit,
            (jnp.zeros((1, _NUM_HEADS), f32), jnp.zeros((1, 1), f32),
             jnp.full((1, 1), _MINIT, f32)))
        bnd_s[:, 0:_NUM_HEADS] = kmax2
        bnd_s[:, _NUM_HEADS:_NUM_HEADS + 1] = smax2
        bnd_s[:, _NUM_HEADS + 1:_NUM_HEADS + 2] = ckmax

    rows = qs + jax.lax.broadcasted_iota(jnp.int32, (1, _QB), 1)
    segq_row = jnp.zeros((1, _QB), jnp.int32)
    for j in range(1, _NSEG):
        segq_row = segq_row + (qp_ref[j] <= rows).astype(jnp.int32)
    seg_q = segq_row.reshape(_QB, 1)

    s0 = jnp.int32(0)
    s1 = jnp.int32(0)
    for j in range(1, _NSEG):
        s0 = s0 + (qp_ref[j] <= qs).astype(jnp.int32)
        s1 = s1 + (qp_ref[j] <= qs + _QB - 1).astype(jnp.int32)
    k_lo = kp_ref[s0]
    k_hi = kp_ref[s1 + 1]
    blo = k_lo // _KB
    bhi = (k_hi + _KB - 1) // _KB

    pqf = (jax.lax.dot_general(phi_q_ref[...].astype(bf16),
                               wa_ref[...].astype(bf16), dn_t,
                               preferred_element_type=f32)
           * _QK_SCALE)
    pq = pqf.astype(bf16)

    sqf = sq_ref[...] * _SIG_SCALE
    sq = sqf.astype(bf16)

    kmax2 = bnd_s[:, 0:_NUM_HEADS]
    smax2 = bnd_s[:, _NUM_HEADS:_NUM_HEADS + 1]
    ckmax = bnd_s[:, _NUM_HEADS + 1:_NUM_HEADS + 2]
    qn2 = jax.lax.dot_general((pqf * pqf).astype(bf16), hsel, dn_m,
                              preferred_element_type=f32)
    sqn2 = jnp.sum(sqf * sqf, axis=1, keepdims=True)
    bb = (jnp.sqrt(qn2 * kmax2) * 1.01
          + jnp.sqrt(sqn2 * smax2) * 1.01 + ckmax)
    bhs = [bb[:, h:h + 1] for h in range(_NUM_HEADS)]

    def body(b, accs):
        koff = b * _KB
        pk = pk_s[pl.ds(koff, _KB), :]
        sk = sk_ref[pl.ds(koff, _KB), :].astype(bf16)
        ck = ck_s[:, pl.ds(koff, _KB)]

        sigdot = jax.lax.dot_general(sq, sk, dn_t,
                                     preferred_element_type=f32)

        cols = koff + jax.lax.broadcasted_iota(jnp.int32, (1, _KB), 1)
        seg_k = jnp.zeros((1, _KB), jnp.int32)
        for j in range(1, _NSEG):
            seg_k = seg_k + (kp_ref[j] <= cols).astype(jnp.int32)
        common = jnp.where(seg_q == seg_k, ck, -jnp.inf) + sigdot

        dots = [jax.lax.dot_general(
            pq[:, h * _HEAD_DIM:(h + 1) * _HEAD_DIM],
            pk[:, h * _HEAD_DIM:(h + 1) * _HEAD_DIM],
            dn_t, preferred_element_type=f32) for h in range(_NUM_HEADS)]

        new_accs = []
        for h in range(_NUM_HEADS):
            p = jnp.exp(common + dots[h] - bhs[h])
            res = jax.lax.dot_general(
                p.astype(bf16), pv_s[pl.ds(koff, _KB), h * _VW:h * _VW + _HD1],
                dn_m, preferred_element_type=f32)
            new_accs.append(accs[h] + res)
        return tuple(new_accs)

    a0 = tuple(jnp.zeros((_QB, _HD1), f32) for _ in range(_NUM_HEADS))
    accs = jax.lax.fori_loop(blo, bhi, body, a0)

    for h in range(_NUM_HEADS):
        sl = slice(h * _HEAD_DIM, (h + 1) * _HEAD_DIM)
        inv = 1.0 / jnp.maximum(accs[h][:, _HEAD_DIM:], 1e-20)
        out_ref[:, sl] = accs[h][:, :_HEAD_DIM] * inv


@functools.partial(jax.jit, static_argnames=("interpret",))
def _run(phi_q, sig_q, q_ptrs, phi_k, sig_k, size_k, k_ptrs, W_A, W_B, W_V,
         interpret=False):
    nq, d = phi_q.shape
    nk = phi_k.shape[0]
    dsig = sig_q.shape[1]
    nqb = nq // _QB
    szk2d = size_k.reshape(1, nk)
    bf16 = jnp.bfloat16

    grid_spec = pltpu.PrefetchScalarGridSpec(
        num_scalar_prefetch=2,
        grid=(nqb,),
        in_specs=[
            pl.BlockSpec((_QB, d), lambda i, qp, kp: (i, 0)),
            pl.BlockSpec((_QB, dsig), lambda i, qp, kp: (i, 0)),
            pl.BlockSpec((nk, d), lambda i, qp, kp: (0, 0)),
            pl.BlockSpec((nk, dsig), lambda i, qp, kp: (0, 0)),
            pl.BlockSpec((1, nk), lambda i, qp, kp: (0, 0)),
            pl.BlockSpec((d, d), lambda i, qp, kp: (0, 0)),
            pl.BlockSpec((d, d), lambda i, qp, kp: (0, 0)),
            pl.BlockSpec((d, d), lambda i, qp, kp: (0, 0)),
        ],
        out_specs=pl.BlockSpec((_QB, d), lambda i, qp, kp: (i, 0)),
        scratch_shapes=[
            pltpu.VMEM((nk, d), bf16),
            pltpu.VMEM((nk, _NUM_HEADS * _VW), bf16),
            pltpu.VMEM((1, nk), jnp.float32),
            pltpu.VMEM((1, 128), jnp.float32),
        ],
    )
    out = pl.pallas_call(
        _fused_body,
        grid_spec=grid_spec,
        out_shape=jax.ShapeDtypeStruct((nq, d), jnp.float32),
        compiler_params=pltpu.CompilerParams(
            dimension_semantics=("arbitrary",)),
        interpret=interpret,
    )(q_ptrs, k_ptrs, phi_q, sig_q, phi_k, sig_k, szk2d, W_A, W_B, W_V)
    return out


def kernel(phi_q, sig_q, size_q, q_ptrs, phi_k, sig_k, size_k, k_ptrs,
           W_A, W_B, W_V):
    out = _run(phi_q, sig_q, q_ptrs, phi_k, sig_k, size_k, k_ptrs,
               W_A, W_B, W_V)
    nq = phi_q.shape[0]
    return (out.reshape(nq, _NUM_HEADS, _HEAD_DIM), q_ptrs)


# speedup vs baseline: 1.3539x; 1.0444x over previous
import functools

import jax
import jax.numpy as jnp
import numpy as np
from jax.experimental import pallas as pl
from jax.experimental.pallas import tpu as pltpu

_D_MODEL = 256
_NUM_HEADS = 4
_HEAD_DIM = _D_MODEL // _NUM_HEADS
_TAU = 1.0
_GAMMA = 0.3
_BETA = 1.0
_ETA = 1.0
_NSEG = 8
_QB = 256
_KB = 512
_MINIT = -1e38
_QK_SCALE = _BETA / np.sqrt(_HEAD_DIM) / _TAU
_SIG_SCALE = 2.0 * _GAMMA / _TAU
_VW = 2 * _HEAD_DIM
_HD1 = _HEAD_DIM + 1


def _fused_body(qp_ref, kp_ref,
                phi_q_ref, sq_ref,
                phi_k_ref, sk_ref,
                szk_ref,
                wa_ref, wb_ref, wv_ref,
                out_ref,
                pk_s, pv_s, ck_s, bnd_s):
    i = pl.program_id(0)
    qs = i * _QB
    f32 = jnp.float32
    bf16 = jnp.bfloat16
    dn_t = (((1,), (1,)), ((), ()))
    dn_m = (((1,), (0,)), ((), ()))
    nk = phi_k_ref.shape[0]

    hrow = jax.lax.broadcasted_iota(jnp.int32, (_D_MODEL, _NUM_HEADS), 0)
    hcol = jax.lax.broadcasted_iota(jnp.int32, (_D_MODEL, _NUM_HEADS), 1)
    hsel = (hrow // _HEAD_DIM == hcol).astype(bf16)

    @pl.when(i == 0)
    def _project_keys():
        wbb = wb_ref[...].astype(bf16)
        wvb = wv_ref[...].astype(bf16)

        def kinit(b, carry):
            kmax2, smax2, ckmax = carry
            koff = b * _KB
            phik = phi_k_ref[pl.ds(koff, _KB), :].astype(bf16)
            pkf = jax.lax.dot_general(phik, wbb, dn_t,
                                      preferred_element_type=f32)
            pk_s[pl.ds(koff, _KB), :] = pkf.astype(bf16)
            kn2 = jax.lax.dot_general((pkf * pkf).astype(bf16), hsel, dn_m,
                                      preferred_element_type=f32)
            kmax2 = jnp.maximum(kmax2, jnp.max(kn2, axis=0, keepdims=True))
            pv = jax.lax.dot_general(
                phik, wvb, dn_t,
                preferred_element_type=f32).astype(bf16)
            ones_blk = jnp.ones((_KB, _HEAD_DIM), bf16)
            for h in range(_NUM_HEADS):
                pv_s[pl.ds(koff, _KB), h * _VW:h * _VW + _HEAD_DIM] = (
                    pv[:, h * _HEAD_DIM:(h + 1) * _HEAD_DIM])
                pv_s[pl.ds(koff, _KB),
                     h * _VW + _HEAD_DIM:(h + 1) * _VW] = ones_blk
            sk = sk_ref[pl.ds(koff, _KB), :]
            sn2 = jnp.sum(sk * sk, axis=1, keepdims=True)
            smax2 = jnp.maximum(smax2, jnp.max(sn2, axis=0, keepdims=True))
            ones_row = jnp.ones((1, sk.shape[1]), f32)
            kn = jax.lax.dot_general(ones_row, sk * sk, dn_t,
                                     preferred_element_type=f32)
            ckrow = (-_GAMMA * kn
                     + _ETA * jnp.log1p(szk_ref[:, pl.ds(koff, _KB)])) / _TAU
            ck_s[:, pl.ds(koff, _KB)] = ckrow
            ckmax = jnp.maximum(ckmax, jnp.max(ckrow, axis=1, keepdims=True))
            return kmax2, smax2, ckmax
        kmax2, smax2, ckmax = jax.lax.fori_loop(
            0, nk // _KB, kinit,
            (jnp.zeros((1, _NUM_HEADS), f32), jnp.zeros((1, 1), f32),
             jnp.full((1, 1), _MINIT, f32)))
        bnd_s[:, 0:_NUM_HEADS] = kmax2
        bnd_s[:, _NUM_HEADS:_NUM_HEADS + 1] = smax2
        bnd_s[:, _NUM_HEADS + 1:_NUM_HEADS + 2] = ckmax

    rows = qs + jax.lax.broadcasted_iota(jnp.int32, (1, _QB), 1)
    segq_row = jnp.zeros((1, _QB), jnp.int32)
    for j in range(1, _NSEG):
        segq_row = segq_row + (qp_ref[j] <= rows).astype(jnp.int32)
    seg_q = segq_row.reshape(_QB, 1)

    s0 = jnp.int32(0)
    s1 = jnp.int32(0)
    for j in range(1, _NSEG):
        s0 = s0 + (qp_ref[j] <= qs).astype(jnp.int32)
        s1 = s1 + (qp_ref[j] <= qs + _QB - 1).astype(jnp.int32)
    k_lo = kp_ref[s0]
    k_hi = kp_ref[s1 + 1]
    blo = k_lo // _KB
    bhi = (k_hi + _KB - 1) // _KB

    pqf = (jax.lax.dot_general(phi_q_ref[...].astype(bf16),
                               wa_ref[...].astype(bf16), dn_t,
                               preferred_element_type=f32)
           * _QK_SCALE)
    pq = pqf.astype(bf16)

    sqf = sq_ref[...] * _SIG_SCALE
    sq = sqf.astype(bf16)

    kmax2 = bnd_s[:, 0:_NUM_HEADS]
    smax2 = bnd_s[:, _NUM_HEADS:_NUM_HEADS + 1]
    ckmax = bnd_s[:, _NUM_HEADS + 1:_NUM_HEADS + 2]
    qn2 = jax.lax.dot_general((pqf * pqf).astype(bf16), hsel, dn_m,
                              preferred_element_type=f32)
    sqn2 = jnp.sum(sqf * sqf, axis=1, keepdims=True)
    bb = (jnp.sqrt(qn2 * kmax2) * 1.01
          + jnp.sqrt(sqn2 * smax2) * 1.01 + ckmax)
    bhs = [bb[:, h:h + 1] for h in range(_NUM_HEADS)]

    def body(b, accs):
        koff = b * _KB
        pk = pk_s[pl.ds(koff, _KB), :]
        sk = sk_ref[pl.ds(koff, _KB), :].astype(bf16)
        ck = ck_s[:, pl.ds(koff, _KB)]

        sigdot = jax.lax.dot_general(sq, sk, dn_t,
                                     preferred_element_type=f32)

        cols = koff + jax.lax.broadcasted_iota(jnp.int32, (1, _KB), 1)
        seg_k = jnp.zeros((1, _KB), jnp.int32)
        for j in range(1, _NSEG):
            seg_k = seg_k + (kp_ref[j] <= cols).astype(jnp.int32)
        common = jnp.where(seg_q == seg_k, ck, -jnp.inf) + sigdot

        dots = [jax.lax.dot_general(
            pq[:, h * _HEAD_DIM:(h + 1) * _HEAD_DIM],
            pk[:, h * _HEAD_DIM:(h + 1) * _HEAD_DIM],
            dn_t, preferred_element_type=f32) for h in range(_NUM_HEADS)]

        new_accs = []
        for h in range(_NUM_HEADS):
            p = jnp.exp(common + dots[h] - bhs[h])
            res = jax.lax.dot_general(
                p.astype(bf16), pv_s[pl.ds(koff, _KB), h * _VW:h * _VW + _HD1],
                dn_m, preferred_element_type=f32)
            new_accs.append(accs[h] + res)
        return tuple(new_accs)

    a0 = tuple(jnp.zeros((_QB, _HD1), f32) for _ in range(_NUM_HEADS))
    accs = jax.lax.fori_loop(blo, bhi, body, a0)

    for h in range(_NUM_HEADS):
        sl = slice(h * _HEAD_DIM, (h + 1) * _HEAD_DIM)
        inv = 1.0 / jnp.maximum(accs[h][:, _HEAD_DIM:], 1e-20)
        out_ref[:, sl] = accs[h][:, :_HEAD_DIM] * inv


@functools.partial(jax.jit, static_argnames=("interpret",))
def _run(phi_q, sig_q, q_ptrs, phi_k, sig_k, size_k, k_ptrs, W_A, W_B, W_V,
         interpret=False):
    nq, d = phi_q.shape
    nk = phi_k.shape[0]
    dsig = sig_q.shape[1]
    nqb = nq // _QB
    szk2d = size_k.reshape(1, nk)
    bf16 = jnp.bfloat16

    grid_spec = pltpu.PrefetchScalarGridSpec(
        num_scalar_prefetch=2,
        grid=(nqb,),
        in_specs=[
            pl.BlockSpec((_QB, d), lambda i, qp, kp: (i, 0)),
            pl.BlockSpec((_QB, dsig), lambda i, qp, kp: (i, 0)),
            pl.BlockSpec((nk, d), lambda i, qp, kp: (0, 0)),
            pl.BlockSpec((nk, dsig), lambda i, qp, kp: (0, 0)),
            pl.BlockSpec((1, nk), lambda i, qp, kp: (0, 0)),
            pl.BlockSpec((d, d), lambda i, qp, kp: (0, 0)),
            pl.BlockSpec((d, d), lambda i, qp, kp: (0, 0)),
            pl.BlockSpec((d, d), lambda i, qp, kp: (0, 0)),
        ],
        out_specs=pl.BlockSpec((_QB, d), lambda i, qp, kp: (i, 0)),
        scratch_shapes=[
            pltpu.VMEM((nk, d), bf16),
            pltpu.VMEM((nk, _NUM_HEADS * _VW), bf16),
            pltpu.VMEM((1, nk), jnp.float32),
            pltpu.VMEM((1, 128), jnp.float32),
        ],
    )
    out = pl.pallas_call(
        _fused_body,
        grid_spec=grid_spec,
        out_shape=jax.ShapeDtypeStruct((nq, d), jnp.float32),
        compiler_params=pltpu.CompilerParams(
            dimension_semantics=("arbitrary",)),
        interpret=interpret,
    )(q_ptrs, k_ptrs, phi_q, sig_q, phi_k, sig_k, szk2d, W_A, W_B, W_V)
    return out


def kernel(phi_q, sig_q, size_q, q_ptrs, phi_k, sig_k, size_k, k_ptrs,
           W_A, W_B, W_V):
    out = _run(phi_q, sig_q, q_ptrs, phi_k, sig_k, size_k, k_ptrs,
               W_A, W_B, W_V)
    nq = phi_q.shape[0]
    return (out.reshape(nq, _NUM_HEADS, _HEAD_DIM), q_ptrs)


# speedup vs baseline: 1.4802x; 1.0933x over previous
import functools

import jax
import jax.numpy as jnp
import numpy as np
from jax.experimental import pallas as pl
from jax.experimental.pallas import tpu as pltpu

_D_MODEL = 256
_NUM_HEADS = 4
_HEAD_DIM = _D_MODEL // _NUM_HEADS
_TAU = 1.0
_GAMMA = 0.3
_BETA = 1.0
_ETA = 1.0
_NSEG = 8
_QB = 512
_KB = 512
_MINIT = -1e38
_QK_SCALE = _BETA / np.sqrt(_HEAD_DIM) / _TAU
_SIG_SCALE = 2.0 * _GAMMA / _TAU
_VW = 2 * _HEAD_DIM
_HD1 = _HEAD_DIM + 1


def _fused_body(qp_ref, kp_ref,
                phi_q_ref, sq_ref,
                phi_k_ref, sk_ref,
                szk_ref,
                wa_ref, wb_ref, wv_ref,
                out_ref,
                pk_s, pv_s, ck_s, bnd_s):
    i = pl.program_id(0)
    qs = i * _QB
    f32 = jnp.float32
    bf16 = jnp.bfloat16
    dn_t = (((1,), (1,)), ((), ()))
    dn_m = (((1,), (0,)), ((), ()))
    nk = phi_k_ref.shape[0]

    hrow = jax.lax.broadcasted_iota(jnp.int32, (_D_MODEL, _NUM_HEADS), 0)
    hcol = jax.lax.broadcasted_iota(jnp.int32, (_D_MODEL, _NUM_HEADS), 1)
    hsel = (hrow // _HEAD_DIM == hcol).astype(bf16)

    @pl.when(i == 0)
    def _project_keys():
        wbb = wb_ref[...].astype(bf16)
        wvb = wv_ref[...].astype(bf16)

        def kinit(b, carry):
            kmax2, smax2, ckmax = carry
            koff = b * _KB
            phik = phi_k_ref[pl.ds(koff, _KB), :].astype(bf16)
            pkf = jax.lax.dot_general(phik, wbb, dn_t,
                                      preferred_element_type=f32)
            pk_s[pl.ds(koff, _KB), :] = pkf.astype(bf16)
            kn2 = jax.lax.dot_general((pkf * pkf).astype(bf16), hsel, dn_m,
                                      preferred_element_type=f32)
            kmax2 = jnp.maximum(kmax2, jnp.max(kn2, axis=0, keepdims=True))
            pv = jax.lax.dot_general(
                phik, wvb, dn_t,
                preferred_element_type=f32).astype(bf16)
            ones_blk = jnp.ones((_KB, _HEAD_DIM), bf16)
            for h in range(_NUM_HEADS):
                pv_s[pl.ds(koff, _KB), h * _VW:h * _VW + _HEAD_DIM] = (
                    pv[:, h * _HEAD_DIM:(h + 1) * _HEAD_DIM])
                pv_s[pl.ds(koff, _KB),
                     h * _VW + _HEAD_DIM:(h + 1) * _VW] = ones_blk
            sk = sk_ref[pl.ds(koff, _KB), :]
            sn2 = jnp.sum(sk * sk, axis=1, keepdims=True)
            smax2 = jnp.maximum(smax2, jnp.max(sn2, axis=0, keepdims=True))
            ones_row = jnp.ones((1, sk.shape[1]), f32)
            kn = jax.lax.dot_general(ones_row, sk * sk, dn_t,
                                     preferred_element_type=f32)
            ckrow = (-_GAMMA * kn
                     + _ETA * jnp.log1p(szk_ref[:, pl.ds(koff, _KB)])) / _TAU
            ck_s[:, pl.ds(koff, _KB)] = ckrow
            ckmax = jnp.maximum(ckmax, jnp.max(ckrow, axis=1, keepdims=True))
            return kmax2, smax2, ckmax
        kmax2, smax2, ckmax = jax.lax.fori_loop(
            0, nk // _KB, kinit,
            (jnp.zeros((1, _NUM_HEADS), f32), jnp.zeros((1, 1), f32),
             jnp.full((1, 1), _MINIT, f32)))
        bnd_s[:, 0:_NUM_HEADS] = kmax2
        bnd_s[:, _NUM_HEADS:_NUM_HEADS + 1] = smax2
        bnd_s[:, _NUM_HEADS + 1:_NUM_HEADS + 2] = ckmax

    rows = qs + jax.lax.broadcasted_iota(jnp.int32, (1, _QB), 1)
    segq_row = jnp.zeros((1, _QB), jnp.int32)
    for j in range(1, _NSEG):
        segq_row = segq_row + (qp_ref[j] <= rows).astype(jnp.int32)
    seg_q = segq_row.reshape(_QB, 1)

    s0 = jnp.int32(0)
    s1 = jnp.int32(0)
    for j in range(1, _NSEG):
        s0 = s0 + (qp_ref[j] <= qs).astype(jnp.int32)
        s1 = s1 + (qp_ref[j] <= qs + _QB - 1).astype(jnp.int32)
    k_lo = kp_ref[s0]
    k_hi = kp_ref[s1 + 1]
    blo = k_lo // _KB
    bhi = (k_hi + _KB - 1) // _KB

    pqf = (jax.lax.dot_general(phi_q_ref[...].astype(bf16),
                               wa_ref[...].astype(bf16), dn_t,
                               preferred_element_type=f32)
           * _QK_SCALE)
    pq = pqf.astype(bf16)

    sqf = sq_ref[...] * _SIG_SCALE
    sq = sqf.astype(bf16)

    kmax2 = bnd_s[:, 0:_NUM_HEADS]
    smax2 = bnd_s[:, _NUM_HEADS:_NUM_HEADS + 1]
    ckmax = bnd_s[:, _NUM_HEADS + 1:_NUM_HEADS + 2]
    qn2 = jax.lax.dot_general((pqf * pqf).astype(bf16), hsel, dn_m,
                              preferred_element_type=f32)
    sqn2 = jnp.sum(sqf * sqf, axis=1, keepdims=True)
    bb = (jnp.sqrt(qn2 * kmax2) * 1.01
          + jnp.sqrt(sqn2 * smax2) * 1.01 + ckmax)
    bhs = [bb[:, h:h + 1] for h in range(_NUM_HEADS)]

    def body(b, accs):
        koff = b * _KB
        pk = pk_s[pl.ds(koff, _KB), :]
        sk = sk_ref[pl.ds(koff, _KB), :].astype(bf16)
        ck = ck_s[:, pl.ds(koff, _KB)]

        sigdot = jax.lax.dot_general(sq, sk, dn_t,
                                     preferred_element_type=f32)

        cols = koff + jax.lax.broadcasted_iota(jnp.int32, (1, _KB), 1)
        seg_k = jnp.zeros((1, _KB), jnp.int32)
        for j in range(1, _NSEG):
            seg_k = seg_k + (kp_ref[j] <= cols).astype(jnp.int32)
        common = jnp.where(seg_q == seg_k, ck, -jnp.inf) + sigdot

        dots = [jax.lax.dot_general(
            pq[:, h * _HEAD_DIM:(h + 1) * _HEAD_DIM],
            pk[:, h * _HEAD_DIM:(h + 1) * _HEAD_DIM],
            dn_t, preferred_element_type=f32) for h in range(_NUM_HEADS)]

        new_accs = []
        for h in range(_NUM_HEADS):
            p = jnp.exp(common + dots[h] - bhs[h])
            res = jax.lax.dot_general(
                p.astype(bf16), pv_s[pl.ds(koff, _KB), h * _VW:h * _VW + _HD1],
                dn_m, preferred_element_type=f32)
            new_accs.append(accs[h] + res)
        return tuple(new_accs)

    a0 = tuple(jnp.zeros((_QB, _HD1), f32) for _ in range(_NUM_HEADS))
    accs = jax.lax.fori_loop(blo, bhi, body, a0)

    for h in range(_NUM_HEADS):
        sl = slice(h * _HEAD_DIM, (h + 1) * _HEAD_DIM)
        inv = 1.0 / jnp.maximum(accs[h][:, _HEAD_DIM:], 1e-20)
        out_ref[:, sl] = accs[h][:, :_HEAD_DIM] * inv


@functools.partial(jax.jit, static_argnames=("interpret",))
def _run(phi_q, sig_q, q_ptrs, phi_k, sig_k, size_k, k_ptrs, W_A, W_B, W_V,
         interpret=False):
    nq, d = phi_q.shape
    nk = phi_k.shape[0]
    dsig = sig_q.shape[1]
    nqb = nq // _QB
    szk2d = size_k.reshape(1, nk)
    bf16 = jnp.bfloat16

    grid_spec = pltpu.PrefetchScalarGridSpec(
        num_scalar_prefetch=2,
        grid=(nqb,),
        in_specs=[
            pl.BlockSpec((_QB, d), lambda i, qp, kp: (i, 0)),
            pl.BlockSpec((_QB, dsig), lambda i, qp, kp: (i, 0)),
            pl.BlockSpec((nk, d), lambda i, qp, kp: (0, 0)),
            pl.BlockSpec((nk, dsig), lambda i, qp, kp: (0, 0)),
            pl.BlockSpec((1, nk), lambda i, qp, kp: (0, 0)),
            pl.BlockSpec((d, d), lambda i, qp, kp: (0, 0)),
            pl.BlockSpec((d, d), lambda i, qp, kp: (0, 0)),
            pl.BlockSpec((d, d), lambda i, qp, kp: (0, 0)),
        ],
        out_specs=pl.BlockSpec((_QB, d), lambda i, qp, kp: (i, 0)),
        scratch_shapes=[
            pltpu.VMEM((nk, d), bf16),
            pltpu.VMEM((nk, _NUM_HEADS * _VW), bf16),
            pltpu.VMEM((1, nk), jnp.float32),
            pltpu.VMEM((1, 128), jnp.float32),
        ],
    )
    out = pl.pallas_call(
        _fused_body,
        grid_spec=grid_spec,
        out_shape=jax.ShapeDtypeStruct((nq, d), jnp.float32),
        compiler_params=pltpu.CompilerParams(
            dimension_semantics=("arbitrary",)),
        interpret=interpret,
    )(q_ptrs, k_ptrs, phi_q, sig_q, phi_k, sig_k, szk2d, W_A, W_B, W_V)
    return out


def kernel(phi_q, sig_q, size_q, q_ptrs, phi_k, sig_k, size_k, k_ptrs,
           W_A, W_B, W_V):
    out = _run(phi_q, sig_q, q_ptrs, phi_k, sig_k, size_k, k_ptrs,
               W_A, W_B, W_V)
    nq = phi_q.shape[0]
    return (out.reshape(nq, _NUM_HEADS, _HEAD_DIM), q_ptrs)
